# baseline (speedup 1.0000x reference)
import functools

import jax
import jax.numpy as jnp
from jax import lax
from jax.experimental import pallas as pl
from jax.experimental.pallas import tpu as pltpu
from jax.experimental.pallas import tpu_sc as plsc

_NUM_CLASSES = 512
_N = 1600000
_M = 100000
_NC = 2
_NS = 16
_NW = _NC * _NS
_PER_W = _N // _NW
_CHUNK = 2000
_N_CHUNKS = _PER_W // _CHUNK
_VREGS = _CHUNK // 16
_LANE_STRIDE = _NUM_CLASSES + 1
_HIST_WORDS = 16 * _LANE_STRIDE
_NBUF = 5


@functools.partial(
    pl.kernel,
    mesh=plsc.VectorSubcoreMesh(core_axis_name="c", subcore_axis_name="s"),
    out_type=jax.ShapeDtypeStruct((_NW, _NUM_CLASSES), jnp.float32),
    compiler_params=pltpu.CompilerParams(needs_layout_passes=False),
    scratch_types=(
        [pltpu.VMEM_SHARED((_M,), jnp.int32)]
        + [pltpu.VMEM((_CHUNK,), jnp.int32) for _ in range(_NBUF)]
        + [pltpu.VMEM((_CHUNK,), jnp.float32) for _ in range(_NBUF)]
        + [pltpu.VMEM((_CHUNK,), jnp.int32) for _ in range(_NBUF)]
        + [pltpu.VMEM((_HIST_WORDS,), jnp.float32),
           pltpu.VMEM((_NUM_CLASSES,), jnp.float32)]
        + [pltpu.SemaphoreType.DMA for _ in range(1 + 3 * _NBUF)]
    ),
)
def _sc_hist(scores_hbm, mgi_hbm, table_hbm, out_hbm, table_sh, *rest):
    idx_bufs = rest[0:_NBUF]
    sc_bufs = rest[_NBUF:2 * _NBUF]
    cls_bufs = rest[2 * _NBUF:3 * _NBUF]
    hist_v = rest[3 * _NBUF]
    out_v = rest[3 * _NBUF + 1]
    sem_t = rest[3 * _NBUF + 2]
    sem_i = rest[3 * _NBUF + 3:3 * _NBUF + 3 + _NBUF]
    sem_s = rest[3 * _NBUF + 3 + _NBUF:3 * _NBUF + 3 + 2 * _NBUF]
    sem_g = rest[3 * _NBUF + 3 + 2 * _NBUF:]

    sid = lax.axis_index("s")
    wid = sid * _NC + lax.axis_index("c")
    base = wid * _PER_W

    @pl.when(sid == 0)
    def _():
        pltpu.async_copy(table_hbm, table_sh, sem_t).wait()

    def start_chunk(ci):
        buf = ci % _NBUF
        off = base + ci * _CHUNK
        i_dma = pltpu.async_copy(
            mgi_hbm.at[pl.ds(off, _CHUNK)], idx_bufs[buf], sem_i[buf])
        s_dma = pltpu.async_copy(
            scores_hbm.at[pl.ds(off, _CHUNK)], sc_bufs[buf], sem_s[buf])
        return i_dma, s_dma

    pending = [start_chunk(ci) for ci in range(_NBUF - 2)]

    zeros16 = jnp.zeros((16,), jnp.float32)

    @plsc.parallel_loop(0, _HIST_WORDS // 16, unroll=9)
    def _(i):
        hist_v[pl.ds(i * 16, 16)] = zeros16

    lane_off = lax.iota(jnp.int32, 16) * _LANE_STRIDE

    plsc.subcore_barrier()

    def start_gather(ci):
        buf = ci % _NBUF
        return pltpu.async_copy(
            table_sh.at[idx_bufs[buf]], cls_bufs[buf], sem_g[buf])

    i0, s0 = pending.pop(0)
    i0.wait()
    g_pending = [(start_gather(0), s0)]

    for ci in range(_N_CHUNKS):
        buf = ci % _NBUF
        if ci + 1 < _N_CHUNKS:
            i_dma, s_dma = pending.pop(0)
            i_dma.wait()
            g_pending.append((start_gather(ci + 1), s_dma))
        if ci + _NBUF - 2 < _N_CHUNKS:
            pending.append(start_chunk(ci + _NBUF - 2))

        g_dma, s_dma = g_pending.pop(0)
        g_dma.wait()
        s_dma.wait()

        cb, sb = cls_bufs[buf], sc_bufs[buf]

        @plsc.parallel_loop(0, _VREGS, unroll=25)
        def _(j):
            cls = cb[pl.ds(j * 16, 16)]
            s = sb[pl.ds(j * 16, 16)]
            plsc.addupdate_scatter(hist_v, [cls + lane_off], s)

    @plsc.parallel_loop(0, _NUM_CLASSES // 16, unroll=4)
    def _(j):
        acc = jnp.zeros((16,), jnp.float32)
        for k in range(16):
            acc = acc + hist_v[pl.ds(k * _LANE_STRIDE + j * 16, 16)]
        out_v[pl.ds(j * 16, 16)] = acc

    pltpu.sync_copy(out_v, out_hbm.at[wid])


def kernel(scores_val, meta_gaussian_indices, meta_gaussian_class_indices):
    hists = _sc_hist(scores_val, meta_gaussian_indices,
                     meta_gaussian_class_indices)
    return jnp.sum(hists, axis=0) * jnp.float32(1.0 / _M)


# speedup vs baseline: 1.1486x; 1.1486x over previous
import functools

import jax
import jax.numpy as jnp
from jax import lax
from jax.experimental import pallas as pl
from jax.experimental.pallas import tpu as pltpu
from jax.experimental.pallas import tpu_sc as plsc

_NUM_CLASSES = 512
_N = 1600000
_M = 100000
_NC = 2
_NS = 16
_NW = _NC * _NS
_PER_W = _N // _NW
_CHUNK = 2000
_N_CHUNKS = _PER_W // _CHUNK
_VREGS = _CHUNK // 16
_LANE_STRIDE = _NUM_CLASSES + 1
_HIST_WORDS = 16 * _LANE_STRIDE
_NBUF = 5


@functools.partial(
    pl.kernel,
    mesh=plsc.VectorSubcoreMesh(core_axis_name="c", subcore_axis_name="s"),
    out_type=jax.ShapeDtypeStruct((_NW, _NUM_CLASSES), jnp.float32),
    compiler_params=pltpu.CompilerParams(needs_layout_passes=False),
    scratch_types=(
        [pltpu.VMEM_SHARED((_M,), jnp.int32)]
        + [pltpu.VMEM((_CHUNK,), jnp.int32) for _ in range(_NBUF)]
        + [pltpu.VMEM((_CHUNK,), jnp.float32) for _ in range(_NBUF)]
        + [pltpu.VMEM((_CHUNK,), jnp.int32) for _ in range(_NBUF)]
        + [pltpu.VMEM((_HIST_WORDS,), jnp.float32),
           pltpu.VMEM((_NUM_CLASSES,), jnp.float32)]
        + [pltpu.SemaphoreType.DMA for _ in range(1 + 3 * _NBUF)]
    ),
)
def _sc_hist(scores_hbm, mgi_hbm, table_hbm, out_hbm, table_sh, *rest):
    idx_bufs = rest[0:_NBUF]
    sc_bufs = rest[_NBUF:2 * _NBUF]
    cls_bufs = rest[2 * _NBUF:3 * _NBUF]
    hist_v = rest[3 * _NBUF]
    out_v = rest[3 * _NBUF + 1]
    sem_t = rest[3 * _NBUF + 2]
    sem_i = rest[3 * _NBUF + 3:3 * _NBUF + 3 + _NBUF]
    sem_s = rest[3 * _NBUF + 3 + _NBUF:3 * _NBUF + 3 + 2 * _NBUF]
    sem_g = rest[3 * _NBUF + 3 + 2 * _NBUF:]

    sid = lax.axis_index("s")
    wid = sid * _NC + lax.axis_index("c")
    base = wid * _PER_W

    @pl.when(sid == 0)
    def _():
        pltpu.async_copy(table_hbm, table_sh, sem_t).wait()

    def start_chunk(ci):
        buf = ci % _NBUF
        off = base + ci * _CHUNK
        i_dma = pltpu.async_copy(
            mgi_hbm.at[pl.ds(off, _CHUNK)], idx_bufs[buf], sem_i[buf])
        s_dma = pltpu.async_copy(
            scores_hbm.at[pl.ds(off, _CHUNK)], sc_bufs[buf], sem_s[buf])
        return i_dma, s_dma

    pending = [start_chunk(ci) for ci in range(_NBUF - 2)]

    zeros16 = jnp.zeros((16,), jnp.float32)

    @plsc.parallel_loop(0, _HIST_WORDS // 16, unroll=9)
    def _(i):
        hist_v[pl.ds(i * 16, 16)] = zeros16

    lane_off = lax.iota(jnp.int32, 16) * _LANE_STRIDE

    plsc.subcore_barrier()

    def start_gather(ci):
        buf = ci % _NBUF
        return pltpu.async_copy(
            table_sh.at[idx_bufs[buf]], cls_bufs[buf], sem_g[buf])

    i0, s0 = pending.pop(0)
    i0.wait()
    g_pending = [(start_gather(0), s0)]

    for ci in range(_N_CHUNKS):
        buf = ci % _NBUF
        if ci + 1 < _N_CHUNKS:
            i_dma, s_dma = pending.pop(0)
            i_dma.wait()
            g_pending.append((start_gather(ci + 1), s_dma))
        if ci + _NBUF - 2 < _N_CHUNKS:
            pending.append(start_chunk(ci + _NBUF - 2))

        g_dma, s_dma = g_pending.pop(0)
        g_dma.wait()
        s_dma.wait()

        cb, sb = cls_bufs[buf], sc_bufs[buf]

        @plsc.parallel_loop(0, _VREGS, unroll=5)
        def _(j):
            cls = cb[pl.ds(j * 16, 16)]
            s = sb[pl.ds(j * 16, 16)]
            plsc.addupdate_scatter(hist_v, [cls + lane_off], s)

    @plsc.parallel_loop(0, _NUM_CLASSES // 16, unroll=4)
    def _(j):
        acc = jnp.zeros((16,), jnp.float32)
        for k in range(16):
            acc = acc + hist_v[pl.ds(k * _LANE_STRIDE + j * 16, 16)]
        out_v[pl.ds(j * 16, 16)] = acc

    pltpu.sync_copy(out_v, out_hbm.at[wid])


def kernel(scores_val, meta_gaussian_indices, meta_gaussian_class_indices):
    hists = _sc_hist(scores_val, meta_gaussian_indices,
                     meta_gaussian_class_indices)
    return jnp.sum(hists, axis=0) * jnp.float32(1.0 / _M)
